# baseline (speedup 1.0000x reference)
import functools

import jax
import jax.numpy as jnp
from jax import lax
from jax.experimental import pallas as pl
from jax.experimental.pallas import tpu as pltpu
from jax.experimental.pallas import tpu_sc as plsc

N = 10000
E = 320000
IN_DIM = 128
HID = 128
OUT = 64

NC = 2
NS = 16
L = 16
NW = NC * NS
EPW = E // NW
CHUNK = 64
NF = EPW // CHUNK
TAIL = EPW - NF * CHUNK
NPAIR = NF // 2

D1 = HID + 8
D2 = OUT + 8

ROWS_PER_TILE = N // NS
ROWB = 10000


def _matmul_bias_body(x_ref, w_ref, cb_ref, o_ref, o2_ref):
    h = jnp.dot(x_ref[...], w_ref[...],
                preferred_element_type=jnp.float32) + cb_ref[...]
    o_ref[...] = h
    o2_ref[...] = jnp.broadcast_to(h[:, HID + 2:HID + 3], (h.shape[0], L))


def _tc_matmul_bias(x, wbig, cb):
    n, k = x.shape
    d = wbig.shape[1]
    return pl.pallas_call(
        _matmul_bias_body,
        grid=(n // ROWB,),
        in_specs=[pl.BlockSpec((ROWB, k), lambda i: (i, 0)),
                  pl.BlockSpec((k, d), lambda i: (0, 0)),
                  pl.BlockSpec((1, d), lambda i: (0, 0))],
        out_specs=[pl.BlockSpec((ROWB, d), lambda i: (i, 0)),
                   pl.BlockSpec((ROWB, L), lambda i: (i, 0))],
        out_shape=[jax.ShapeDtypeStruct((n, d), jnp.float32),
                   jax.ShapeDtypeStruct((n, L), jnp.float32)],
    )(x, wbig, cb)


def _combine(ap, hab, b, c):
    s = ap[0, :, :c] + ap[1, :, :c]
    d = ap[0, :, c:c + 1] + ap[1, :, c:c + 1]
    h = hab[:, :c]
    asad = hab[:, c + 1:c + 2] + hab[:, c + 2:c + 3]
    wself = jnp.exp(jnp.where(asad >= 0, asad, 0.2 * asad))
    num = s + wself * h
    den = jnp.maximum(d + wself, 1e-16)
    return num / den + b


def _prep2_body(ap_ref, hab_ref, b_ref, w2_ref, cb2_ref, o_ref, o2_ref):
    out1 = _combine(ap_ref[...], hab_ref[...], b_ref[...], HID)
    x2 = jnp.maximum(out1, 0.0)
    h2 = jnp.dot(x2, w2_ref[...],
                 preferred_element_type=jnp.float32) + cb2_ref[...]
    o_ref[...] = h2
    o2_ref[...] = jnp.broadcast_to(h2[:, OUT + 2:OUT + 3], (h2.shape[0], L))


def _tc_prep2(ap, hab, b1, wbig2, cb2):
    return pl.pallas_call(
        _prep2_body,
        grid=(N // ROWB,),
        in_specs=[pl.BlockSpec((NC, ROWB, D1), lambda i: (0, i, 0)),
                  pl.BlockSpec((ROWB, D1), lambda i: (i, 0)),
                  pl.BlockSpec((1, HID), lambda i: (0, 0)),
                  pl.BlockSpec((HID, D2), lambda i: (0, 0)),
                  pl.BlockSpec((1, D2), lambda i: (0, 0))],
        out_specs=[pl.BlockSpec((ROWB, D2), lambda i: (i, 0)),
                   pl.BlockSpec((ROWB, L), lambda i: (i, 0))],
        out_shape=[jax.ShapeDtypeStruct((N, D2), jnp.float32),
                   jax.ShapeDtypeStruct((N, L), jnp.float32)],
    )(ap, hab, b1, wbig2, cb2)


def _final_body(ap_ref, hab_ref, b_ref, o_ref):
    out2 = _combine(ap_ref[...], hab_ref[...], b_ref[...], OUT)
    m = jnp.max(out2, axis=1, keepdims=True)
    z = out2 - m
    lse = jnp.log(jnp.sum(jnp.exp(z), axis=1, keepdims=True))
    o_ref[...] = z - lse


def _tc_final(ap, hab, b2):
    return pl.pallas_call(
        _final_body,
        grid=(N // ROWB,),
        in_specs=[pl.BlockSpec((NC, ROWB, D2), lambda i: (0, i, 0)),
                  pl.BlockSpec((ROWB, D2), lambda i: (i, 0)),
                  pl.BlockSpec((1, OUT), lambda i: (0, 0))],
        out_specs=pl.BlockSpec((ROWB, OUT), lambda i: (i, 0)),
        out_shape=jax.ShapeDtypeStruct((N, OUT), jnp.float32),
    )(ap, hab, b2)


def _make_sc_aggregate(D, C):
    NVF = C // L
    mesh = plsc.VectorSubcoreMesh(core_axis_name="c", subcore_axis_name="s",
                                  num_cores=NC, num_subcores=NS)

    def body(h_hbm, eidx_hbm, adp_hbm, out_hbm,
             accum_sh, sidx_v, didx_v, sca, dca, scb, dcb, sct, dct,
             rows_a, rows_b, adp_a, adp_b, semga, semgb, semsa, semsb):
        cid = lax.axis_index("c")
        sid = lax.axis_index("s")
        wid = sid * NC + cid
        base = wid * EPW

        pltpu.async_copy(eidx_hbm.at[0, pl.ds(base, EPW)], sidx_v, semga)
        pltpu.async_copy(eidx_hbm.at[1, pl.ds(base, EPW)], didx_v, semgb)

        zoffs = [v * L for v in range(NVF)] + [C - 8]
        for r in range(CHUNK):
            for o in zoffs:
                rows_a[r, pl.ds(o, L)] = jnp.zeros((L,), jnp.float32)
        r0 = sid * ROWS_PER_TILE
        n_full = ROWS_PER_TILE // CHUNK
        rem = ROWS_PER_TILE - n_full * CHUNK

        def zcopy(k, c):
            pltpu.sync_copy(rows_a, accum_sh.at[pl.ds(r0 + k * CHUNK, CHUNK)])
            return c
        lax.fori_loop(0, n_full, zcopy, 0)
        if rem:
            pltpu.sync_copy(rows_a.at[pl.ds(0, rem)],
                            accum_sh.at[pl.ds(r0 + n_full * CHUNK, rem)])
        pltpu.make_async_copy(eidx_hbm.at[0, pl.ds(base, EPW)], sidx_v,
                              semga).wait()
        pltpu.make_async_copy(eidx_hbm.at[1, pl.ds(base, EPW)], didx_v,
                              semgb).wait()
        plsc.subcore_barrier()

        ar16 = jnp.arange(L, dtype=jnp.int32)
        cas = jnp.full((L,), C + 1, jnp.int32)
        c0 = jnp.zeros((L,), jnp.int32)

        def prep_idx(sb, db, c):
            off = pl.multiple_of(c * CHUNK, 16)
            for g in range(CHUNK // L):
                sb[pl.ds(g * L, L)] = sidx_v[pl.ds(off + g * L, L)]
                db[pl.ds(g * L, L)] = didx_v[pl.ds(off + g * L, L)]

        def gathers(sb, db, rb, ab, sem):
            pltpu.async_copy(h_hbm.at[sb], rb, sem)
            pltpu.async_copy(adp_hbm.at[db], ab, sem)

        def wait_gathers(sb, db, rb, ab, sem):
            pltpu.make_async_copy(h_hbm.at[sb], rb, sem).wait()
            pltpu.make_async_copy(adp_hbm.at[db], ab, sem).wait()

        mask8 = ar16 < 8

        def process(rb, ab, ngroups):
            for g in range(ngroups):
                ridx = ar16 + (g * L)
                a = (plsc.load_gather(rb, [ridx, cas]) +
                     plsc.load_gather(ab, [ridx, c0]))
                a = jnp.where(a >= 0, a, jnp.float32(0.2) * a)
                w16 = jnp.exp(a)
                for i in range(L):
                    wr = w16[i]
                    r = g * L + i
                    for v in range(NVF):
                        rb[r, pl.ds(v * L, L)] = rb[r, pl.ds(v * L, L)] * wr
                    tw = jnp.where(mask8, jnp.float32(1.0), wr)
                    rb[r, pl.ds(C - 8, L)] = rb[r, pl.ds(C - 8, L)] * tw

        prep_idx(sca, dca, 0)
        gathers(sca, dca, rows_a, adp_a, semga)

        def pair(m, c):
            cA = 2 * m
            cB = cA + 1
            prep_idx(scb, dcb, cB)
            gathers(scb, dcb, rows_b, adp_b, semgb)
            wait_gathers(sca, dca, rows_a, adp_a, semga)
            process(rows_a, adp_a, CHUNK // L)
            pltpu.async_copy(rows_a, accum_sh.at[dca], semsa, add=True)
            wait_gathers(scb, dcb, rows_b, adp_b, semgb)
            process(rows_b, adp_b, CHUNK // L)
            pltpu.async_copy(rows_b, accum_sh.at[dcb], semsb, add=True)
            pltpu.make_async_copy(rows_a, accum_sh.at[dca], semsa).wait()

            @pl.when(m < NPAIR - 1)
            def _():
                prep_idx(sca, dca, cA + 2)
                gathers(sca, dca, rows_a, adp_a, semga)

            pltpu.make_async_copy(rows_b, accum_sh.at[dcb], semsb).wait()
            return c
        lax.fori_loop(0, NPAIR, pair, 0)

        if TAIL:
            toff = NF * CHUNK
            sct[pl.ds(0, L)] = sidx_v[pl.ds(toff, L)]
            dct[pl.ds(0, L)] = didx_v[pl.ds(toff, L)]
            pltpu.async_copy(h_hbm.at[sct], rows_a.at[pl.ds(0, TAIL)], semga)
            pltpu.make_async_copy(h_hbm.at[sct], rows_a.at[pl.ds(0, TAIL)],
                                  semga).wait()
            pltpu.async_copy(adp_hbm.at[dct], adp_a.at[pl.ds(0, TAIL)], semga)
            pltpu.make_async_copy(adp_hbm.at[dct], adp_a.at[pl.ds(0, TAIL)],
                                  semga).wait()
            process(rows_a, adp_a, TAIL // L)
            pltpu.sync_copy(rows_a.at[pl.ds(0, TAIL)], accum_sh.at[dct],
                            add=True)

        plsc.subcore_barrier()
        pltpu.sync_copy(accum_sh.at[pl.ds(r0, ROWS_PER_TILE)],
                        out_hbm.at[cid, pl.ds(r0, ROWS_PER_TILE)])

    return pl.kernel(
        body,
        out_type=jax.ShapeDtypeStruct((NC, N, D), jnp.float32),
        mesh=mesh,
        scratch_types=[
            pltpu.VMEM_SHARED((N, D), jnp.float32),
            pltpu.VMEM((EPW,), jnp.int32),
            pltpu.VMEM((EPW,), jnp.int32),
            pltpu.VMEM((CHUNK,), jnp.int32),
            pltpu.VMEM((CHUNK,), jnp.int32),
            pltpu.VMEM((CHUNK,), jnp.int32),
            pltpu.VMEM((CHUNK,), jnp.int32),
            pltpu.VMEM((L,), jnp.int32),
            pltpu.VMEM((L,), jnp.int32),
            pltpu.VMEM((CHUNK, D), jnp.float32),
            pltpu.VMEM((CHUNK, D), jnp.float32),
            pltpu.VMEM((CHUNK, L), jnp.float32),
            pltpu.VMEM((CHUNK, L), jnp.float32),
            pltpu.SemaphoreType.DMA,
            pltpu.SemaphoreType.DMA,
            pltpu.SemaphoreType.DMA,
            pltpu.SemaphoreType.DMA,
        ],
        compiler_params=pltpu.CompilerParams(use_tc_tiling_on_sc=False,
                                             needs_layout_passes=False),
    )


_sc_aggregate = functools.cache(_make_sc_aggregate)


def _build_wbig(W, a_s, a_d, d_total):
    k, c = W.shape
    pad = d_total - c - 3
    return jnp.concatenate(
        [W, jnp.zeros((k, 1), jnp.float32), (W @ a_s)[:, None],
         (W @ a_d)[:, None], jnp.zeros((k, pad), jnp.float32)], axis=1)


def kernel(x, edge_index, W1, a_src1, a_dst1, b1, W2, a_src2, a_dst2, b2):
    wbig1 = _build_wbig(W1, a_src1.reshape(HID), a_dst1.reshape(HID), D1)
    cb1 = jnp.zeros((1, D1), jnp.float32).at[0, HID].set(1.0)
    hab1, adp1 = _tc_matmul_bias(x, wbig1, cb1)
    part1 = _sc_aggregate(D1, HID)(hab1, edge_index, adp1)

    wbig2 = _build_wbig(W2, a_src2.reshape(OUT), a_dst2.reshape(OUT), D2)
    cb2 = jnp.zeros((1, D2), jnp.float32).at[0, OUT].set(1.0)
    hab2, adp2 = _tc_prep2(part1, hab1,
                           b1.reshape(1, HID), wbig2, cb2)
    part2 = _sc_aggregate(D2, OUT)(hab2, edge_index, adp2)

    return _tc_final(part2, hab2, b2.reshape(1, OUT))


# speedup vs baseline: 1.0153x; 1.0153x over previous
import functools

import jax
import jax.numpy as jnp
from jax import lax
from jax.experimental import pallas as pl
from jax.experimental.pallas import tpu as pltpu
from jax.experimental.pallas import tpu_sc as plsc

N = 10000
E = 320000
IN_DIM = 128
HID = 128
OUT = 64

NC = 2
NS = 16
L = 16
NW = NC * NS
EPW = E // NW
CHUNK = 64
NF = EPW // CHUNK
TAIL = EPW - NF * CHUNK
NPAIR = NF // 2

D1 = HID + 8
D2 = OUT + 8

ROWS_PER_TILE = N // NS
ROWB = 2000


def _matmul_bias_body(x_ref, w_ref, cb_ref, o_ref, o2_ref):
    h = jnp.dot(x_ref[...], w_ref[...],
                preferred_element_type=jnp.float32) + cb_ref[...]
    o_ref[...] = h
    o2_ref[...] = jnp.broadcast_to(h[:, HID + 2:HID + 3], (h.shape[0], L))


def _tc_matmul_bias(x, wbig, cb):
    n, k = x.shape
    d = wbig.shape[1]
    return pl.pallas_call(
        _matmul_bias_body,
        grid=(n // ROWB,),
        in_specs=[pl.BlockSpec((ROWB, k), lambda i: (i, 0)),
                  pl.BlockSpec((k, d), lambda i: (0, 0)),
                  pl.BlockSpec((1, d), lambda i: (0, 0))],
        out_specs=[pl.BlockSpec((ROWB, d), lambda i: (i, 0)),
                   pl.BlockSpec((ROWB, L), lambda i: (i, 0))],
        out_shape=[jax.ShapeDtypeStruct((n, d), jnp.float32),
                   jax.ShapeDtypeStruct((n, L), jnp.float32)],
    )(x, wbig, cb)


def _combine(ap, hab, b, c):
    s = ap[0, :, :c] + ap[1, :, :c]
    d = ap[0, :, c:c + 1] + ap[1, :, c:c + 1]
    h = hab[:, :c]
    asad = hab[:, c + 1:c + 2] + hab[:, c + 2:c + 3]
    wself = jnp.exp(jnp.where(asad >= 0, asad, 0.2 * asad))
    num = s + wself * h
    den = jnp.maximum(d + wself, 1e-16)
    return num / den + b


def _prep2_body(ap_ref, hab_ref, b_ref, w2_ref, cb2_ref, o_ref, o2_ref):
    out1 = _combine(ap_ref[...], hab_ref[...], b_ref[...], HID)
    x2 = jnp.maximum(out1, 0.0)
    h2 = jnp.dot(x2, w2_ref[...],
                 preferred_element_type=jnp.float32) + cb2_ref[...]
    o_ref[...] = h2
    o2_ref[...] = jnp.broadcast_to(h2[:, OUT + 2:OUT + 3], (h2.shape[0], L))


def _tc_prep2(ap, hab, b1, wbig2, cb2):
    return pl.pallas_call(
        _prep2_body,
        grid=(N // ROWB,),
        in_specs=[pl.BlockSpec((NC, ROWB, D1), lambda i: (0, i, 0)),
                  pl.BlockSpec((ROWB, D1), lambda i: (i, 0)),
                  pl.BlockSpec((1, HID), lambda i: (0, 0)),
                  pl.BlockSpec((HID, D2), lambda i: (0, 0)),
                  pl.BlockSpec((1, D2), lambda i: (0, 0))],
        out_specs=[pl.BlockSpec((ROWB, D2), lambda i: (i, 0)),
                   pl.BlockSpec((ROWB, L), lambda i: (i, 0))],
        out_shape=[jax.ShapeDtypeStruct((N, D2), jnp.float32),
                   jax.ShapeDtypeStruct((N, L), jnp.float32)],
    )(ap, hab, b1, wbig2, cb2)


def _final_body(ap_ref, hab_ref, b_ref, o_ref):
    out2 = _combine(ap_ref[...], hab_ref[...], b_ref[...], OUT)
    m = jnp.max(out2, axis=1, keepdims=True)
    z = out2 - m
    lse = jnp.log(jnp.sum(jnp.exp(z), axis=1, keepdims=True))
    o_ref[...] = z - lse


def _tc_final(ap, hab, b2):
    return pl.pallas_call(
        _final_body,
        grid=(N // ROWB,),
        in_specs=[pl.BlockSpec((NC, ROWB, D2), lambda i: (0, i, 0)),
                  pl.BlockSpec((ROWB, D2), lambda i: (i, 0)),
                  pl.BlockSpec((1, OUT), lambda i: (0, 0))],
        out_specs=pl.BlockSpec((ROWB, OUT), lambda i: (i, 0)),
        out_shape=jax.ShapeDtypeStruct((N, OUT), jnp.float32),
    )(ap, hab, b2)


def _make_sc_aggregate(D, C):
    NVF = C // L
    mesh = plsc.VectorSubcoreMesh(core_axis_name="c", subcore_axis_name="s",
                                  num_cores=NC, num_subcores=NS)

    def body(h_hbm, eidx_hbm, adp_hbm, out_hbm,
             accum_sh, sidx_v, didx_v, sca, dca, scb, dcb, sct, dct,
             rows_a, rows_b, adp_a, adp_b, semga, semgb, semsa, semsb):
        cid = lax.axis_index("c")
        sid = lax.axis_index("s")
        wid = sid * NC + cid
        base = wid * EPW

        pltpu.async_copy(eidx_hbm.at[0, pl.ds(base, EPW)], sidx_v, semga)
        pltpu.async_copy(eidx_hbm.at[1, pl.ds(base, EPW)], didx_v, semgb)

        zoffs = [v * L for v in range(NVF)] + [C - 8]
        for r in range(CHUNK):
            for o in zoffs:
                rows_a[r, pl.ds(o, L)] = jnp.zeros((L,), jnp.float32)
        r0 = sid * ROWS_PER_TILE
        n_full = ROWS_PER_TILE // CHUNK
        rem = ROWS_PER_TILE - n_full * CHUNK

        def zcopy(k, c):
            pltpu.sync_copy(rows_a, accum_sh.at[pl.ds(r0 + k * CHUNK, CHUNK)])
            return c
        lax.fori_loop(0, n_full, zcopy, 0)
        if rem:
            pltpu.sync_copy(rows_a.at[pl.ds(0, rem)],
                            accum_sh.at[pl.ds(r0 + n_full * CHUNK, rem)])
        pltpu.make_async_copy(eidx_hbm.at[0, pl.ds(base, EPW)], sidx_v,
                              semga).wait()
        pltpu.make_async_copy(eidx_hbm.at[1, pl.ds(base, EPW)], didx_v,
                              semgb).wait()
        plsc.subcore_barrier()

        ar16 = jnp.arange(L, dtype=jnp.int32)
        cas = jnp.full((L,), C + 1, jnp.int32)
        c0 = jnp.zeros((L,), jnp.int32)

        def prep_idx(sb, db, c):
            off = pl.multiple_of(c * CHUNK, 16)
            for g in range(CHUNK // L):
                sb[pl.ds(g * L, L)] = sidx_v[pl.ds(off + g * L, L)]
                db[pl.ds(g * L, L)] = didx_v[pl.ds(off + g * L, L)]

        def gathers(sb, db, rb, ab, sem):
            pltpu.async_copy(h_hbm.at[sb], rb, sem)
            pltpu.async_copy(adp_hbm.at[db], ab, sem)

        def wait_gathers(sb, db, rb, ab, sem):
            pltpu.make_async_copy(h_hbm.at[sb], rb, sem).wait()
            pltpu.make_async_copy(adp_hbm.at[db], ab, sem).wait()

        mask8 = ar16 < 8

        def process(rb, ab, ngroups):
            for g in range(ngroups):
                ridx = ar16 + (g * L)
                a = (plsc.load_gather(rb, [ridx, cas]) +
                     plsc.load_gather(ab, [ridx, c0]))
                a = jnp.where(a >= 0, a, jnp.float32(0.2) * a)
                w16 = jnp.exp(a)
                for i in range(L):
                    wr = w16[i]
                    r = g * L + i
                    for v in range(NVF):
                        rb[r, pl.ds(v * L, L)] = rb[r, pl.ds(v * L, L)] * wr
                    tw = jnp.where(mask8, jnp.float32(1.0), wr)
                    rb[r, pl.ds(C - 8, L)] = rb[r, pl.ds(C - 8, L)] * tw

        prep_idx(sca, dca, 0)
        gathers(sca, dca, rows_a, adp_a, semga)

        def pair(m, c):
            cA = 2 * m
            cB = cA + 1
            prep_idx(scb, dcb, cB)
            gathers(scb, dcb, rows_b, adp_b, semgb)
            wait_gathers(sca, dca, rows_a, adp_a, semga)
            process(rows_a, adp_a, CHUNK // L)
            pltpu.async_copy(rows_a, accum_sh.at[dca], semsa, add=True)
            wait_gathers(scb, dcb, rows_b, adp_b, semgb)
            process(rows_b, adp_b, CHUNK // L)
            pltpu.async_copy(rows_b, accum_sh.at[dcb], semsb, add=True)
            pltpu.make_async_copy(rows_a, accum_sh.at[dca], semsa).wait()

            @pl.when(m < NPAIR - 1)
            def _():
                prep_idx(sca, dca, cA + 2)
                gathers(sca, dca, rows_a, adp_a, semga)

            pltpu.make_async_copy(rows_b, accum_sh.at[dcb], semsb).wait()
            return c
        lax.fori_loop(0, NPAIR, pair, 0)

        if TAIL:
            toff = NF * CHUNK
            sct[pl.ds(0, L)] = sidx_v[pl.ds(toff, L)]
            dct[pl.ds(0, L)] = didx_v[pl.ds(toff, L)]
            pltpu.async_copy(h_hbm.at[sct], rows_a.at[pl.ds(0, TAIL)], semga)
            pltpu.make_async_copy(h_hbm.at[sct], rows_a.at[pl.ds(0, TAIL)],
                                  semga).wait()
            pltpu.async_copy(adp_hbm.at[dct], adp_a.at[pl.ds(0, TAIL)], semga)
            pltpu.make_async_copy(adp_hbm.at[dct], adp_a.at[pl.ds(0, TAIL)],
                                  semga).wait()
            process(rows_a, adp_a, TAIL // L)
            pltpu.sync_copy(rows_a.at[pl.ds(0, TAIL)], accum_sh.at[dct],
                            add=True)

        plsc.subcore_barrier()
        pltpu.sync_copy(accum_sh.at[pl.ds(r0, ROWS_PER_TILE)],
                        out_hbm.at[cid, pl.ds(r0, ROWS_PER_TILE)])

    return pl.kernel(
        body,
        out_type=jax.ShapeDtypeStruct((NC, N, D), jnp.float32),
        mesh=mesh,
        scratch_types=[
            pltpu.VMEM_SHARED((N, D), jnp.float32),
            pltpu.VMEM((EPW,), jnp.int32),
            pltpu.VMEM((EPW,), jnp.int32),
            pltpu.VMEM((CHUNK,), jnp.int32),
            pltpu.VMEM((CHUNK,), jnp.int32),
            pltpu.VMEM((CHUNK,), jnp.int32),
            pltpu.VMEM((CHUNK,), jnp.int32),
            pltpu.VMEM((L,), jnp.int32),
            pltpu.VMEM((L,), jnp.int32),
            pltpu.VMEM((CHUNK, D), jnp.float32),
            pltpu.VMEM((CHUNK, D), jnp.float32),
            pltpu.VMEM((CHUNK, L), jnp.float32),
            pltpu.VMEM((CHUNK, L), jnp.float32),
            pltpu.SemaphoreType.DMA,
            pltpu.SemaphoreType.DMA,
            pltpu.SemaphoreType.DMA,
            pltpu.SemaphoreType.DMA,
        ],
        compiler_params=pltpu.CompilerParams(use_tc_tiling_on_sc=False,
                                             needs_layout_passes=False),
    )


_sc_aggregate = functools.cache(_make_sc_aggregate)


def _build_wbig(W, a_s, a_d, d_total):
    k, c = W.shape
    pad = d_total - c - 3
    return jnp.concatenate(
        [W, jnp.zeros((k, 1), jnp.float32), (W @ a_s)[:, None],
         (W @ a_d)[:, None], jnp.zeros((k, pad), jnp.float32)], axis=1)


def kernel(x, edge_index, W1, a_src1, a_dst1, b1, W2, a_src2, a_dst2, b2):
    wbig1 = _build_wbig(W1, a_src1.reshape(HID), a_dst1.reshape(HID), D1)
    cb1 = jnp.zeros((1, D1), jnp.float32).at[0, HID].set(1.0)
    hab1, adp1 = _tc_matmul_bias(x, wbig1, cb1)
    part1 = _sc_aggregate(D1, HID)(hab1, edge_index, adp1)

    wbig2 = _build_wbig(W2, a_src2.reshape(OUT), a_dst2.reshape(OUT), D2)
    cb2 = jnp.zeros((1, D2), jnp.float32).at[0, OUT].set(1.0)
    hab2, adp2 = _tc_prep2(part1, hab1,
                           b1.reshape(1, HID), wbig2, cb2)
    part2 = _sc_aggregate(D2, OUT)(hab2, edge_index, adp2)

    return _tc_final(part2, hab2, b2.reshape(1, OUT))
